# baseline (speedup 1.0000x reference)
import functools

import jax
import jax.numpy as jnp
from jax.experimental import pallas as pl
from jax.experimental.pallas import tpu as pltpu

_FAULT_TYPE = 6


def _lstm_kernel(C, TB, T, F, H,
                 loc_ref, xT_ref, wc_ref, wo_ref, bo_ref, out_ref):
    del loc_ref
    wc = wc_ref[...]
    ones_row = jnp.ones((1, TB), jnp.float32)

    def step(t, carry):
        off = pl.multiple_of(t * F, 8)
        new = []
        for ci in range(C):
            h, c = carry[ci]
            x_t = xT_ref[pl.ds(off, F), ci * TB:(ci + 1) * TB]
            buf = jnp.concatenate([x_t, h, ones_row], axis=0)
            gates = jnp.dot(wc, buf, preferred_element_type=jnp.float32)
            tg = jnp.tanh(gates)
            sig = tg[:3 * H] * 0.5 + 0.5
            i_g = sig[0 * H:1 * H]
            f_g = sig[1 * H:2 * H]
            o_g = sig[2 * H:3 * H]
            g_g = tg[3 * H:]
            c2 = f_g * c + i_g * g_g
            h2 = o_g * jnp.tanh(c2)
            new.append((h2, c2))
        return tuple(new)

    zeros = jnp.zeros((H, TB), jnp.float32)
    init = tuple((zeros, zeros) for _ in range(C))
    final = jax.lax.fori_loop(0, T, step, init)

    wo = wo_ref[...]
    bo = bo_ref[...]
    for ci in range(C):
        h_last, _ = final[ci]
        out_ref[:, ci * TB:(ci + 1) * TB] = (
            jnp.dot(wo, h_last, preferred_element_type=jnp.float32) + bo)


@jax.jit
def _forward(loc_idx, x, k_wih, k_whh, k_bg, k_wo, k_bo):
    B, T, F = x.shape
    H = k_whh.shape[1]
    G = 4 * H
    CP = k_wo.shape[1]

    TB = 256
    C = 4
    CB = C * TB
    while CB > B and TB > 128:
        TB //= 2
        CB = C * TB
    Bp = ((B + CB - 1) // CB) * CB
    nb = Bp // CB

    x2 = x.reshape(B, T * F).T
    if Bp != B:
        x2 = jnp.pad(x2, ((0, 0), (0, Bp - B)))

    scale = jnp.concatenate([jnp.full((3 * H, 1), 0.5, jnp.float32),
                             jnp.ones((H, 1), jnp.float32)])
    w_cat = jnp.concatenate([k_wih, k_whh, k_bg], axis=1) * scale

    grid_spec = pltpu.PrefetchScalarGridSpec(
        num_scalar_prefetch=1,
        grid=(nb,),
        in_specs=[
            pl.BlockSpec((T * F, CB), lambda b, loc: (0, b)),
            pl.BlockSpec((G, F + H + 1), lambda b, loc: (0, 0)),
            pl.BlockSpec((None, CP, H), lambda b, loc: (loc[0], 0, 0)),
            pl.BlockSpec((None, CP, 1), lambda b, loc: (loc[0], 0, 0)),
        ],
        out_specs=pl.BlockSpec((CP, CB), lambda b, loc: (0, b)),
    )

    out_p = pl.pallas_call(
        functools.partial(_lstm_kernel, C, TB, T, F, H),
        out_shape=jax.ShapeDtypeStruct((CP, Bp), jnp.float32),
        grid_spec=grid_spec,
        compiler_params=pltpu.CompilerParams(
            dimension_semantics=("parallel",),
            vmem_limit_bytes=48 << 20,
        ),
    )(loc_idx, x2, w_cat, k_wo, k_bo)

    return out_p[:_FAULT_TYPE, :B].T


def kernel(loc_idx, x, k_wih, k_whh, k_bg, k_wo, k_bo):
    return _forward(loc_idx, x, k_wih, k_whh, k_bg, k_wo, k_bo)


# speedup vs baseline: 1.0500x; 1.0500x over previous
import functools

import jax
import jax.numpy as jnp
from jax.experimental import pallas as pl
from jax.experimental.pallas import tpu as pltpu

_FAULT_TYPE = 6


def _lstm_kernel(C, TB, T, F, H,
                 loc_ref, xT_ref, wc_ref, wo_ref, bo_ref, out_ref):
    del loc_ref
    wc = wc_ref[...]
    ones_row = jnp.ones((1, TB), jnp.float32)

    def step(t, carry):
        off = pl.multiple_of(t * F, 8)
        new = []
        for ci in range(C):
            h, c = carry[ci]
            x_t = xT_ref[pl.ds(off, F), ci * TB:(ci + 1) * TB]
            buf = jnp.concatenate([x_t, h, ones_row], axis=0)
            gates = jnp.dot(wc, buf, preferred_element_type=jnp.float32)
            tg = jnp.tanh(gates)
            sig = tg[:3 * H] * 0.5 + 0.5
            i_g = sig[0 * H:1 * H]
            f_g = sig[1 * H:2 * H]
            o_g = sig[2 * H:3 * H]
            g_g = tg[3 * H:]
            c2 = f_g * c + i_g * g_g
            h2 = o_g * jnp.tanh(c2)
            new.append((h2, c2))
        return tuple(new)

    zeros = jnp.zeros((H, TB), jnp.float32)
    init = tuple((zeros, zeros) for _ in range(C))
    final = jax.lax.fori_loop(0, T, step, init, unroll=2)

    wo = wo_ref[...]
    bo = bo_ref[...]
    for ci in range(C):
        h_last, _ = final[ci]
        out_ref[:, ci * TB:(ci + 1) * TB] = (
            jnp.dot(wo, h_last, preferred_element_type=jnp.float32) + bo)


@jax.jit
def _forward(loc_idx, x, k_wih, k_whh, k_bg, k_wo, k_bo):
    B, T, F = x.shape
    H = k_whh.shape[1]
    G = 4 * H
    CP = k_wo.shape[1]

    TB = 256
    C = 4
    CB = C * TB
    while CB > B and TB > 128:
        TB //= 2
        CB = C * TB
    Bp = ((B + CB - 1) // CB) * CB
    nb = Bp // CB

    x2 = x.reshape(B, T * F).T
    if Bp != B:
        x2 = jnp.pad(x2, ((0, 0), (0, Bp - B)))

    scale = jnp.concatenate([jnp.full((3 * H, 1), 0.5, jnp.float32),
                             jnp.ones((H, 1), jnp.float32)])
    w_cat = jnp.concatenate([k_wih, k_whh, k_bg], axis=1) * scale

    grid_spec = pltpu.PrefetchScalarGridSpec(
        num_scalar_prefetch=1,
        grid=(nb,),
        in_specs=[
            pl.BlockSpec((T * F, CB), lambda b, loc: (0, b)),
            pl.BlockSpec((G, F + H + 1), lambda b, loc: (0, 0)),
            pl.BlockSpec((None, CP, H), lambda b, loc: (loc[0], 0, 0)),
            pl.BlockSpec((None, CP, 1), lambda b, loc: (loc[0], 0, 0)),
        ],
        out_specs=pl.BlockSpec((CP, CB), lambda b, loc: (0, b)),
    )

    out_p = pl.pallas_call(
        functools.partial(_lstm_kernel, C, TB, T, F, H),
        out_shape=jax.ShapeDtypeStruct((CP, Bp), jnp.float32),
        grid_spec=grid_spec,
        compiler_params=pltpu.CompilerParams(
            dimension_semantics=("parallel",),
            vmem_limit_bytes=48 << 20,
        ),
    )(loc_idx, x2, w_cat, k_wo, k_bo)

    return out_p[:_FAULT_TYPE, :B].T


def kernel(loc_idx, x, k_wih, k_whh, k_bg, k_wo, k_bo):
    return _forward(loc_idx, x, k_wih, k_whh, k_bg, k_wo, k_bo)


# speedup vs baseline: 1.3533x; 1.2889x over previous
import functools

import jax
import jax.numpy as jnp
from jax.experimental import pallas as pl
from jax.experimental.pallas import tpu as pltpu

_FAULT_TYPE = 6


def _lstm_kernel(C, TB, T, F, H,
                 loc_ref, xT_ref, wc_ref, wo_ref, bo_ref, out_ref):
    del loc_ref
    wc = wc_ref[...]
    ones_row = jnp.ones((1, TB), jnp.float32)

    def step(t, carry):
        off = pl.multiple_of(t * F, 8)
        new = []
        for ci in range(C):
            h, c = carry[ci]
            x_t = xT_ref[pl.ds(off, F), ci * TB:(ci + 1) * TB]
            buf = jnp.concatenate([x_t, h, ones_row], axis=0)
            gates = jnp.dot(wc, buf, preferred_element_type=jnp.float32)
            tg = jnp.tanh(gates)
            sig = tg[:3 * H] * 0.5 + 0.5
            i_g = sig[0 * H:1 * H]
            f_g = sig[1 * H:2 * H]
            o_g = sig[2 * H:3 * H]
            g_g = tg[3 * H:]
            c2 = f_g * c + i_g * g_g
            h2 = o_g * jnp.tanh(c2)
            new.append((h2, c2))
        return tuple(new)

    zeros = jnp.zeros((H, TB), jnp.float32)
    init = tuple((zeros, zeros) for _ in range(C))
    final = jax.lax.fori_loop(0, T, step, init)

    wo = wo_ref[...]
    bo = bo_ref[...]
    for ci in range(C):
        h_last, _ = final[ci]
        out_ref[:, ci * TB:(ci + 1) * TB] = (
            jnp.dot(wo, h_last, preferred_element_type=jnp.float32) + bo)


@jax.jit
def _forward(loc_idx, x, k_wih, k_whh, k_bg, k_wo, k_bo):
    B, T, F = x.shape
    H = k_whh.shape[1]
    G = 4 * H
    CP = k_wo.shape[1]

    TB = 2048
    C = 1
    CB = C * TB
    while CB > B and TB > 128:
        TB //= 2
        CB = C * TB
    Bp = ((B + CB - 1) // CB) * CB
    nb = Bp // CB

    x2 = x.reshape(B, T * F).T
    if Bp != B:
        x2 = jnp.pad(x2, ((0, 0), (0, Bp - B)))

    scale = jnp.concatenate([jnp.full((3 * H, 1), 0.5, jnp.float32),
                             jnp.ones((H, 1), jnp.float32)])
    w_cat = jnp.concatenate([k_wih, k_whh, k_bg], axis=1) * scale

    grid_spec = pltpu.PrefetchScalarGridSpec(
        num_scalar_prefetch=1,
        grid=(nb,),
        in_specs=[
            pl.BlockSpec((T * F, CB), lambda b, loc: (0, b)),
            pl.BlockSpec((G, F + H + 1), lambda b, loc: (0, 0)),
            pl.BlockSpec((None, CP, H), lambda b, loc: (loc[0], 0, 0)),
            pl.BlockSpec((None, CP, 1), lambda b, loc: (loc[0], 0, 0)),
        ],
        out_specs=pl.BlockSpec((CP, CB), lambda b, loc: (0, b)),
    )

    out_p = pl.pallas_call(
        functools.partial(_lstm_kernel, C, TB, T, F, H),
        out_shape=jax.ShapeDtypeStruct((CP, Bp), jnp.float32),
        grid_spec=grid_spec,
        compiler_params=pltpu.CompilerParams(
            dimension_semantics=("parallel",),
            vmem_limit_bytes=48 << 20,
        ),
    )(loc_idx, x2, w_cat, k_wo, k_bo)

    return out_p[:_FAULT_TYPE, :B].T


def kernel(loc_idx, x, k_wih, k_whh, k_bg, k_wo, k_bo):
    return _forward(loc_idx, x, k_wih, k_whh, k_bg, k_wo, k_bo)


# speedup vs baseline: 1.5926x; 1.1768x over previous
import functools

import jax
import jax.numpy as jnp
from jax.experimental import pallas as pl
from jax.experimental.pallas import tpu as pltpu

_FAULT_TYPE = 6


def _lstm_kernel(C, TB, T, F, H,
                 loc_ref, xT_ref, wc_ref, wo_ref, bo_ref, out_ref):
    del loc_ref
    wc = wc_ref[...]
    ones_row = jnp.ones((1, TB), jnp.float32)

    def step(t, carry):
        off = pl.multiple_of(t * F, 8)
        new = []
        for ci in range(C):
            h, c = carry[ci]
            x_t = xT_ref[pl.ds(off, F), ci * TB:(ci + 1) * TB]
            buf = jnp.concatenate([x_t, h, ones_row], axis=0)
            gates = jnp.dot(wc, buf, preferred_element_type=jnp.float32)
            tg = jnp.tanh(gates)
            sig = tg[:3 * H] * 0.5 + 0.5
            i_g = sig[0 * H:1 * H]
            f_g = sig[1 * H:2 * H]
            o_g = sig[2 * H:3 * H]
            g_g = tg[3 * H:]
            c2 = f_g * c + i_g * g_g
            h2 = o_g * jnp.tanh(c2)
            new.append((h2, c2))
        return tuple(new)

    zeros = jnp.zeros((H, TB), jnp.float32)
    init = tuple((zeros, zeros) for _ in range(C))
    final = jax.lax.fori_loop(0, T, step, init)

    wo = wo_ref[...]
    bo = bo_ref[...]
    for ci in range(C):
        h_last, _ = final[ci]
        out_ref[:, ci * TB:(ci + 1) * TB] = (
            jnp.dot(wo, h_last, preferred_element_type=jnp.float32) + bo)


@jax.jit
def _forward(loc_idx, x, k_wih, k_whh, k_bg, k_wo, k_bo):
    B, T, F = x.shape
    H = k_whh.shape[1]
    G = 4 * H
    CP = k_wo.shape[1]

    TB = 4096
    C = 1
    CB = C * TB
    while CB > B and TB > 128:
        TB //= 2
        CB = C * TB
    Bp = ((B + CB - 1) // CB) * CB
    nb = Bp // CB

    x2 = x.reshape(B, T * F).T
    if Bp != B:
        x2 = jnp.pad(x2, ((0, 0), (0, Bp - B)))

    scale = jnp.concatenate([jnp.full((3 * H, 1), 0.5, jnp.float32),
                             jnp.ones((H, 1), jnp.float32)])
    w_cat = jnp.concatenate([k_wih, k_whh, k_bg], axis=1) * scale

    grid_spec = pltpu.PrefetchScalarGridSpec(
        num_scalar_prefetch=1,
        grid=(nb,),
        in_specs=[
            pl.BlockSpec((T * F, CB), lambda b, loc: (0, b)),
            pl.BlockSpec((G, F + H + 1), lambda b, loc: (0, 0)),
            pl.BlockSpec((None, CP, H), lambda b, loc: (loc[0], 0, 0)),
            pl.BlockSpec((None, CP, 1), lambda b, loc: (loc[0], 0, 0)),
        ],
        out_specs=pl.BlockSpec((CP, CB), lambda b, loc: (0, b)),
    )

    out_p = pl.pallas_call(
        functools.partial(_lstm_kernel, C, TB, T, F, H),
        out_shape=jax.ShapeDtypeStruct((CP, Bp), jnp.float32),
        grid_spec=grid_spec,
        compiler_params=pltpu.CompilerParams(
            dimension_semantics=("parallel",),
            vmem_limit_bytes=48 << 20,
        ),
    )(loc_idx, x2, w_cat, k_wo, k_bo)

    return out_p[:_FAULT_TYPE, :B].T


def kernel(loc_idx, x, k_wih, k_whh, k_bg, k_wo, k_bo):
    return _forward(loc_idx, x, k_wih, k_whh, k_bg, k_wo, k_bo)
